# Initial kernel scaffold; baseline (speedup 1.0000x reference)
import jax
import jax.numpy as jnp
from jax.experimental import pallas as pl


def kernel(node_features, edge_features, W1, b1, gamma1, beta1, W2, b2):
    raise NotImplementedError("write your pallas kernel here")


# speedup vs baseline: 8.6300x; 8.6300x over previous
import functools

import jax
import jax.numpy as jnp
from jax.experimental import pallas as pl
from jax.experimental.pallas import tpu as pltpu

N = 32
F = 128
D_E = 16
H = 256
EMB = 128
E_REAL = N * (N - 1)
BB = 8


def _body(nf_ref, ef_ref, ws_ref, wd_ref, we_ref, b1_ref, w2_ref, b2_ref,
          out_ref, full_ref):
    nf = nf_ref[...].reshape(BB * N, F)
    p = jnp.dot(nf, ws_ref[...], preferred_element_type=jnp.float32)
    p = p + b1_ref[...]
    q = jnp.dot(nf, wd_ref[...], preferred_element_type=jnp.float32)

    full_ref[...] = jnp.zeros((BB, N * N, D_E), dtype=jnp.float32)
    for m in range(N - 1):
        full_ref[:, 33 * m + 1:33 * m + 33, :] = ef_ref[:, 32 * m:32 * m + 32, :]
    ec = jnp.dot(full_ref[...].reshape(BB * N * N, D_E), we_ref[...],
                 preferred_element_type=jnp.float32)

    z = (ec.reshape(BB, N, N, H)
         + p.reshape(BB, N, 1, H)
         + q.reshape(BB, 1, N, H))
    s = jnp.maximum(z, 0.0).sum(axis=2).reshape(BB * N, H)

    out = jnp.dot(s, w2_ref[...], preferred_element_type=jnp.float32)
    out = jnp.maximum(out + b2_ref[...], 0.0)
    out_ref[...] = out.reshape(BB, N, EMB)


@functools.partial(jax.jit, static_argnames=())
def kernel(node_features, edge_features, W1, b1, gamma1, beta1, W2, b2):
    B = node_features.shape[0]
    g = jax.lax.rsqrt(jnp.float32(1.0 + 1e-3)) * gamma1
    w1g = W1 * g[None, :]
    ws = w1g[:F]
    wd = w1g[F:2 * F]
    we = w1g[2 * F:]
    b1g = (b1 * g + beta1).reshape(1, H)
    b2t = (jnp.float32(N) * b2).reshape(1, EMB)

    grid = (B // BB,)
    out = pl.pallas_call(
        _body,
        grid=grid,
        in_specs=[
            pl.BlockSpec((BB, N, F), lambda b: (b, 0, 0)),
            pl.BlockSpec((BB, E_REAL, D_E), lambda b: (b, 0, 0)),
            pl.BlockSpec((F, H), lambda b: (0, 0)),
            pl.BlockSpec((F, H), lambda b: (0, 0)),
            pl.BlockSpec((D_E, H), lambda b: (0, 0)),
            pl.BlockSpec((1, H), lambda b: (0, 0)),
            pl.BlockSpec((H, EMB), lambda b: (0, 0)),
            pl.BlockSpec((1, EMB), lambda b: (0, 0)),
        ],
        out_specs=pl.BlockSpec((BB, N, EMB), lambda b: (b, 0, 0)),
        out_shape=jax.ShapeDtypeStruct((B, N, EMB), jnp.float32),
        scratch_shapes=[pltpu.VMEM((BB, N * N, D_E), jnp.float32)],
        compiler_params=pltpu.CompilerParams(
            dimension_semantics=("arbitrary",)),
    )(node_features, edge_features, ws, wd, we, b1g, W2, b2t)
    return out
